# Initial kernel scaffold; baseline (speedup 1.0000x reference)
import jax
import jax.numpy as jnp
from jax.experimental import pallas as pl


def kernel(x, edge_index, W1, b1, W2, b2):
    raise NotImplementedError("write your pallas kernel here")


# speedup vs baseline: 13.2445x; 13.2445x over previous
import functools

import jax
import jax.numpy as jnp
from jax import lax
from jax.experimental import pallas as pl
from jax.experimental.pallas import tpu as pltpu
from jax.experimental.pallas import tpu_sc as plsc

N = 10000
D = 128
NPAD = 10240
E = 320000
NC, NS = 2, 16
NW = NC * NS
EPT = NPAD
EPAD = NW * EPT
CH = 128
CPT = EPT // CH
RPT = NPAD // NS

_MESH = plsc.VectorSubcoreMesh(core_axis_name="c", subcore_axis_name="s")


@functools.partial(
    pl.kernel,
    out_type=jax.ShapeDtypeStruct((NC, NPAD), jnp.float32),
    mesh=_MESH,
    scratch_types=[
        pltpu.VMEM_SHARED((NPAD,), jnp.float32),
        pltpu.VMEM((CH,), jnp.int32),
        pltpu.VMEM((CH,), jnp.float32),
    ],
)
def _deg_kernel(dst_hbm, zeros1_hbm, deg_out, deg_acc, dst_v, ones_v):
    c = lax.axis_index("c")
    s = lax.axis_index("s")
    pltpu.sync_copy(zeros1_hbm.at[pl.ds(s * RPT, RPT)],
                    deg_acc.at[pl.ds(s * RPT, RPT)])
    for i in range(CH // 16):
        ones_v[pl.ds(i * 16, 16)] = jnp.ones((16,), jnp.float32)
    plsc.subcore_barrier()
    base = (c * NS + s) * EPT

    def body(i, carry):
        pltpu.sync_copy(dst_hbm.at[pl.ds(base + i * CH, CH)], dst_v)
        pltpu.sync_copy(ones_v, deg_acc.at[dst_v], add=True)
        return carry

    lax.fori_loop(0, CPT, body, 0)
    plsc.subcore_barrier()
    pltpu.sync_copy(deg_acc.at[pl.ds(s * RPT, RPT)],
                    deg_out.at[c, pl.ds(s * RPT, RPT)])


@functools.partial(
    pl.kernel,
    out_type=jax.ShapeDtypeStruct((NC, NPAD, D), jnp.float32),
    mesh=_MESH,
    scratch_types=[
        pltpu.VMEM_SHARED((NPAD, D), jnp.float32),
        pltpu.VMEM((CH,), jnp.int32),
        pltpu.VMEM((CH,), jnp.int32),
        pltpu.VMEM((CH, D), jnp.float32),
    ],
)
def _agg_kernel(g_hbm, src_hbm, dst_hbm, zeros2_hbm, s_out,
                acc, src_v, dst_v, rows_v):
    c = lax.axis_index("c")
    s = lax.axis_index("s")
    pltpu.sync_copy(zeros2_hbm.at[pl.ds(s * RPT, RPT)],
                    acc.at[pl.ds(s * RPT, RPT)])
    plsc.subcore_barrier()
    base = (c * NS + s) * EPT

    def body(i, carry):
        off = base + i * CH
        pltpu.sync_copy(src_hbm.at[pl.ds(off, CH)], src_v)
        pltpu.sync_copy(dst_hbm.at[pl.ds(off, CH)], dst_v)
        pltpu.sync_copy(g_hbm.at[src_v], rows_v)
        pltpu.sync_copy(rows_v, acc.at[dst_v], add=True)
        return carry

    lax.fori_loop(0, CPT, body, 0)
    plsc.subcore_barrier()
    pltpu.sync_copy(acc.at[pl.ds(s * RPT, RPT)],
                    s_out.at[c, pl.ds(s * RPT, RPT)])


def _dinv_rowscale(degp_ref):
    deg = degp_ref[0, 0, :] + degp_ref[0, 1, :] + 1.0
    dinv = lax.rsqrt(jnp.maximum(deg, 1.0))
    return lax.dot_general(
        dinv.reshape(1, D), jnp.ones((1, D), jnp.float32),
        (((0,), (0,)), ((), ())),
        preferred_element_type=jnp.float32,
        precision=lax.Precision.HIGHEST)


def _tc_first_body(x_ref, w_ref, degp_ref, g_ref):
    dmat = _dinv_rowscale(degp_ref)
    h = jnp.dot(x_ref[...], w_ref[...],
                preferred_element_type=jnp.float32,
                precision=lax.Precision.HIGHEST)
    g_ref[...] = h * dmat


def _tc_mid_body(sp_ref, g_ref, degp_ref, w_ref, b_ref, g2_ref):
    dmat = _dinv_rowscale(degp_ref)
    ssum = sp_ref[0] + sp_ref[1] + g_ref[...]
    h = jnp.maximum(ssum * dmat + b_ref[0, :], 0.0)
    g2_ref[...] = jnp.dot(h, w_ref[...],
                          preferred_element_type=jnp.float32,
                          precision=lax.Precision.HIGHEST) * dmat


def _tc_last_body(sp_ref, g_ref, degp_ref, b_ref, out_ref):
    dmat = _dinv_rowscale(degp_ref)
    ssum = sp_ref[0] + sp_ref[1] + g_ref[...]
    out_ref[...] = ssum * dmat + b_ref[0, :]


_GRID = NPAD // 128
_BLK = pl.BlockSpec((128, D), lambda i: (i, 0))
_BLK_SP = pl.BlockSpec((NC, 128, D), lambda i: (0, i, 0))
_BLK_DEG = pl.BlockSpec((1, NC, D), lambda i: (i, 0, 0))
_BLK_W = pl.BlockSpec((D, D), lambda i: (0, 0))
_BLK_B = pl.BlockSpec((1, D), lambda i: (0, 0))
_OUT2D = jax.ShapeDtypeStruct((NPAD, D), jnp.float32)

_tc_first = pl.pallas_call(
    _tc_first_body, grid=(_GRID,),
    in_specs=[_BLK, _BLK_W, _BLK_DEG],
    out_specs=_BLK, out_shape=_OUT2D)

_tc_mid = pl.pallas_call(
    _tc_mid_body, grid=(_GRID,),
    in_specs=[_BLK_SP, _BLK, _BLK_DEG, _BLK_W, _BLK_B],
    out_specs=_BLK, out_shape=_OUT2D)

_tc_last = pl.pallas_call(
    _tc_last_body, grid=(_GRID,),
    in_specs=[_BLK_SP, _BLK, _BLK_DEG, _BLK_B],
    out_specs=_BLK, out_shape=_OUT2D)


def kernel(x, edge_index, W1, b1, W2, b2):
    src = edge_index[0]
    dst = edge_index[1]
    pad = jnp.arange(EPAD - E, dtype=jnp.int32)
    src_p = jnp.concatenate([src, pad % N])
    dst_p = jnp.concatenate([dst, N + pad % (NPAD - N)])
    x_p = jnp.concatenate(
        [x, jnp.zeros((NPAD - N, D), jnp.float32)], axis=0)
    zeros1 = jnp.zeros((NPAD,), jnp.float32)
    zeros2 = jnp.zeros((NPAD, D), jnp.float32)

    degp = _deg_kernel(dst_p, zeros1)
    degp3 = degp.reshape(NC, _GRID, D).transpose(1, 0, 2)

    g1 = _tc_first(x_p, W1, degp3)
    s1p = _agg_kernel(g1, src_p, dst_p, zeros2)
    g2 = _tc_mid(s1p, g1, degp3, W2, b1.reshape(1, D))
    s2p = _agg_kernel(g2, src_p, dst_p, zeros2)
    out = _tc_last(s2p, g2, degp3, b2.reshape(1, D))
    return out[:N]


# speedup vs baseline: 21.3723x; 1.6137x over previous
import functools

import jax
import jax.numpy as jnp
from jax import lax
from jax.experimental import pallas as pl
from jax.experimental.pallas import tpu as pltpu
from jax.experimental.pallas import tpu_sc as plsc

N = 10000
D = 128
NPAD = 10240
E = 320000
NC, NS = 2, 16
NW = NC * NS
EPT = NPAD
EPAD = NW * EPT
CH = 128
CPT = EPT // CH
RPT = NPAD // NS

_MESH = plsc.VectorSubcoreMesh(core_axis_name="c", subcore_axis_name="s")


KD = 8
DGPT = (EPT // CH) // KD


@functools.partial(
    pl.kernel,
    out_type=jax.ShapeDtypeStruct((NC, NPAD), jnp.float32),
    mesh=_MESH,
    scratch_types=[
        pltpu.VMEM_SHARED((NPAD,), jnp.float32),
        pltpu.VMEM((2, KD, CH), jnp.int32),
        pltpu.VMEM((CH,), jnp.float32),
        pltpu.SemaphoreType.DMA,
        pltpu.SemaphoreType.DMA,
    ],
)
def _deg_kernel(edge_hbm, zeros1_hbm, deg_out, deg_acc, dst_v, ones_v,
                dsem0, dsem1):
    c = lax.axis_index("c")
    s = lax.axis_index("s")
    dsems = (dsem0, dsem1)
    pltpu.sync_copy(zeros1_hbm.at[pl.ds(s * RPT, RPT)],
                    deg_acc.at[pl.ds(s * RPT, RPT)])
    for i in range(CH // 16):
        ones_v[pl.ds(i * 16, 16)] = jnp.ones((16,), jnp.float32)
    base = (c * NS + s) * (EPT // CH)

    def load_idx(b, grp):
        pltpu.sync_copy(edge_hbm.at[1, pl.ds(base + grp * KD, KD)],
                        dst_v.at[b])

    def scatter_ones(b):
        descs = [pltpu.async_copy(ones_v, deg_acc.at[dst_v.at[b, j]],
                                  dsems[b], add=True)
                 for j in range(KD)]
        for d in descs:
            d.wait()

    plsc.subcore_barrier()
    for b in range(2):
        load_idx(b, b)

    def body(i, carry):
        for b in range(2):
            scatter_ones(b)
            load_idx(b, i * 2 + b + 2)
        return carry

    lax.fori_loop(0, DGPT // 2 - 1, body, 0)
    for b in range(2):
        scatter_ones(b)
    plsc.subcore_barrier()
    pltpu.sync_copy(deg_acc.at[pl.ds(s * RPT, RPT)],
                    deg_out.at[c, pl.ds(s * RPT, RPT)])


NBUF = 2
CROWS = EPT // CH


@functools.partial(
    pl.kernel,
    out_type=jax.ShapeDtypeStruct((NC, NPAD, D), jnp.float32),
    mesh=_MESH,
    scratch_types=[
        pltpu.VMEM_SHARED((NPAD, D), jnp.float32),
        pltpu.VMEM((NBUF, 2, CH), jnp.int32),
        pltpu.VMEM((NBUF, CH, D), jnp.float32),
        pltpu.SemaphoreType.DMA,
        pltpu.SemaphoreType.DMA,
    ],
)
def _agg_kernel(g_hbm, edge_hbm, zeros2_hbm, s_out,
                acc, idx_v, rows_v, gsem0, gsem1):
    c = lax.axis_index("c")
    s = lax.axis_index("s")
    gsems = (gsem0, gsem1)
    pltpu.sync_copy(zeros2_hbm.at[pl.ds(s * RPT, RPT)],
                    acc.at[pl.ds(s * RPT, RPT)])
    base = (c * NS + s) * CROWS

    def load_and_fire(b, chunk):
        pltpu.sync_copy(edge_hbm.at[:, base + chunk], idx_v.at[b])
        pltpu.async_copy(g_hbm.at[idx_v.at[b, 0]], rows_v.at[b], gsems[b])

    def drain_gather(b):
        pltpu.make_async_copy(g_hbm.at[idx_v.at[b, 0]],
                              rows_v.at[b], gsems[b]).wait()

    def scatter(b):
        pltpu.sync_copy(rows_v.at[b], acc.at[idx_v.at[b, 1]], add=True)

    plsc.subcore_barrier()

    for b in range(NBUF):
        load_and_fire(b, b)

    def body(i, carry):
        for b in range(NBUF):
            drain_gather(b)
            scatter(b)
            load_and_fire(b, i * NBUF + b + NBUF)
        return carry

    lax.fori_loop(0, CROWS // NBUF - 1, body, 0)
    for b in range(NBUF):
        drain_gather(b)
        scatter(b)

    plsc.subcore_barrier()
    pltpu.sync_copy(acc.at[pl.ds(s * RPT, RPT)],
                    s_out.at[c, pl.ds(s * RPT, RPT)])


def _dinv_rowscale(degp_ref):
    deg = degp_ref[0, 0, :] + degp_ref[0, 1, :] + 1.0
    dinv = lax.rsqrt(jnp.maximum(deg, 1.0))
    return lax.dot_general(
        dinv.reshape(1, D), jnp.ones((1, D), jnp.float32),
        (((0,), (0,)), ((), ())),
        preferred_element_type=jnp.float32,
        precision=lax.Precision.HIGHEST)


def _tc_first_body(x_ref, w_ref, degp_ref, g_ref):
    dmat = _dinv_rowscale(degp_ref)
    h = jnp.dot(x_ref[...], w_ref[...],
                preferred_element_type=jnp.float32,
                precision=lax.Precision.HIGHEST)
    g_ref[...] = h * dmat


def _tc_mid_body(sp_ref, g_ref, degp_ref, w_ref, b_ref, g2_ref):
    dmat = _dinv_rowscale(degp_ref)
    ssum = sp_ref[0] + sp_ref[1] + g_ref[...]
    h = jnp.maximum(ssum * dmat + b_ref[0, :], 0.0)
    g2_ref[...] = jnp.dot(h, w_ref[...],
                          preferred_element_type=jnp.float32,
                          precision=lax.Precision.HIGHEST) * dmat


def _tc_last_body(sp_ref, g_ref, degp_ref, b_ref, out_ref):
    dmat = _dinv_rowscale(degp_ref)
    ssum = sp_ref[0] + sp_ref[1] + g_ref[...]
    out_ref[...] = ssum * dmat + b_ref[0, :]


_GRID = NPAD // 128
_BLK = pl.BlockSpec((128, D), lambda i: (i, 0))
_BLK_SP = pl.BlockSpec((NC, 128, D), lambda i: (0, i, 0))
_BLK_DEG = pl.BlockSpec((1, NC, D), lambda i: (i, 0, 0))
_BLK_W = pl.BlockSpec((D, D), lambda i: (0, 0))
_BLK_B = pl.BlockSpec((1, D), lambda i: (0, 0))
_OUT2D = jax.ShapeDtypeStruct((NPAD, D), jnp.float32)

_tc_first = pl.pallas_call(
    _tc_first_body, grid=(_GRID,),
    in_specs=[_BLK, _BLK_W, _BLK_DEG],
    out_specs=_BLK, out_shape=_OUT2D)

_tc_mid = pl.pallas_call(
    _tc_mid_body, grid=(_GRID,),
    in_specs=[_BLK_SP, _BLK, _BLK_DEG, _BLK_W, _BLK_B],
    out_specs=_BLK, out_shape=_OUT2D)

_tc_last = pl.pallas_call(
    _tc_last_body, grid=(_GRID,),
    in_specs=[_BLK_SP, _BLK, _BLK_DEG, _BLK_B],
    out_specs=_BLK, out_shape=_OUT2D)


def kernel(x, edge_index, W1, b1, W2, b2):
    src = edge_index[0]
    dst = edge_index[1]
    pad = jnp.arange(EPAD - E, dtype=jnp.int32)
    src_p = jnp.concatenate([src, pad % N])
    dst_p = jnp.concatenate([dst, N + pad % (NPAD - N)])
    edge2d = jnp.stack([src_p, dst_p]).reshape(2, NW * CROWS, CH)
    x_p = jnp.concatenate(
        [x, jnp.zeros((NPAD - N, D), jnp.float32)], axis=0)
    zeros1 = jnp.zeros((NPAD,), jnp.float32)
    zeros2 = jnp.zeros((NPAD, D), jnp.float32)

    degp = _deg_kernel(edge2d, zeros1)
    degp3 = degp.reshape(NC, _GRID, D).transpose(1, 0, 2)

    g1 = _tc_first(x_p, W1, degp3)
    s1p = _agg_kernel(g1, edge2d, zeros2)
    g2 = _tc_mid(s1p, g1, degp3, W2, b1.reshape(1, D))
    s2p = _agg_kernel(g2, edge2d, zeros2)
    out = _tc_last(s2p, g2, degp3, b2.reshape(1, D))
    return out[:N]


# speedup vs baseline: 31.3376x; 1.4663x over previous
import functools

import jax
import jax.numpy as jnp
from jax import lax
from jax.experimental import pallas as pl
from jax.experimental.pallas import tpu as pltpu
from jax.experimental.pallas import tpu_sc as plsc

N = 10000
D = 128
NPAD = 10240
E = 320000
NC, NS = 2, 16
NW = NC * NS
CH = 128
EROWS = E // CH
RPW = EROWS // NW
XROWS = EROWS - RPW * NW
RPT = NPAD // NS
NACC = 10112
APT = NACC // NS

_MESH = plsc.VectorSubcoreMesh(core_axis_name="c", subcore_axis_name="s")


KD = 6
DG = RPW // KD
assert DG * KD == RPW


@functools.partial(
    pl.kernel,
    out_type=jax.ShapeDtypeStruct((NC, NPAD), jnp.float32),
    mesh=_MESH,
    scratch_types=[
        pltpu.VMEM_SHARED((NPAD,), jnp.float32),
        pltpu.VMEM((2, KD, CH), jnp.int32),
        pltpu.VMEM((CH,), jnp.float32),
        pltpu.SemaphoreType.DMA,
        pltpu.SemaphoreType.DMA,
        pltpu.SemaphoreType.DMA,
        pltpu.SemaphoreType.DMA,
    ],
)
def _deg_kernel(edge_hbm, zeros1_hbm, deg_out, deg_acc, dst_v, ones_v,
                dsem0, dsem1, lsem0, lsem1):
    c = lax.axis_index("c")
    s = lax.axis_index("s")
    w = c * NS + s
    dsems = (dsem0, dsem1)
    lsems = (lsem0, lsem1)
    pltpu.sync_copy(zeros1_hbm.at[pl.ds(s * RPT, RPT)],
                    deg_acc.at[pl.ds(s * RPT, RPT)])
    for i in range(CH // 16):
        ones_v[pl.ds(i * 16, 16)] = jnp.ones((16,), jnp.float32)
    base = w * RPW

    def load_idx(b, grp):
        for j in range(KD):
            pltpu.async_copy(edge_hbm.at[1, base + grp * KD + j],
                             dst_v.at[b, j], lsems[b])

    def drain_idx(b, grp):
        for j in range(KD):
            pltpu.make_async_copy(edge_hbm.at[1, base + grp * KD + j],
                                  dst_v.at[b, j], lsems[b]).wait()

    def scatter_ones(b, grp):
        drain_idx(b, grp)
        descs = [pltpu.async_copy(ones_v, deg_acc.at[dst_v.at[b, j]],
                                  dsems[b], add=True)
                 for j in range(KD)]
        for d in descs:
            d.wait()

    plsc.subcore_barrier()
    for b in range(2):
        load_idx(b, b)

    def body(i, carry):
        for b in range(2):
            scatter_ones(b, i * 2 + b)
            load_idx(b, i * 2 + b + 2)
        return carry

    lax.fori_loop(0, DG // 2 - 1, body, 0)
    scatter_ones(0, DG - 3)
    load_idx(0, DG - 1)
    scatter_ones(1, DG - 2)
    scatter_ones(0, DG - 1)

    @pl.when(w == NW - 1)
    def _():
        for k in range(XROWS):
            pltpu.sync_copy(edge_hbm.at[1, NW * RPW + k], dst_v.at[0, 0])
            pltpu.sync_copy(ones_v, deg_acc.at[dst_v.at[0, 0]], add=True)

    plsc.subcore_barrier()
    pltpu.sync_copy(deg_acc.at[pl.ds(s * RPT, RPT)],
                    deg_out.at[c, pl.ds(s * RPT, RPT)])


NBUF = 3


@functools.partial(
    pl.kernel,
    out_type=jax.ShapeDtypeStruct((NC, NACC, D), jnp.float32),
    mesh=_MESH,
    scratch_types=[
        pltpu.VMEM_SHARED((NACC, D), jnp.float32),
        pltpu.VMEM((NBUF, 2, CH), jnp.int32),
        pltpu.VMEM((NBUF, CH, D), jnp.float32),
        pltpu.SemaphoreType.DMA,
        pltpu.SemaphoreType.DMA,
        pltpu.SemaphoreType.DMA,
    ],
)
def _agg_kernel(g_hbm, edge_hbm, zeros2_hbm, s_out,
                acc, idx_v, rows_v, gsem0, gsem1, gsem2):
    c = lax.axis_index("c")
    s = lax.axis_index("s")
    w = c * NS + s
    gsems = (gsem0, gsem1, gsem2)
    pltpu.sync_copy(zeros2_hbm.at[pl.ds(s * APT, APT)],
                    acc.at[pl.ds(s * APT, APT)])
    base = w * RPW

    def load_and_fire(b, chunk):
        pltpu.sync_copy(edge_hbm.at[:, chunk], idx_v.at[b])
        pltpu.async_copy(g_hbm.at[idx_v.at[b, 0]], rows_v.at[b], gsems[b])

    def drain_gather(b):
        pltpu.make_async_copy(g_hbm.at[idx_v.at[b, 0]],
                              rows_v.at[b], gsems[b]).wait()

    def scatter(b):
        pltpu.sync_copy(rows_v.at[b], acc.at[idx_v.at[b, 1]], add=True)

    plsc.subcore_barrier()

    for b in range(NBUF):
        load_and_fire(b, base + b)

    def body(i, carry):
        for b in range(NBUF):
            drain_gather(b)
            scatter(b)
            load_and_fire(b, base + i * NBUF + b + NBUF)
        return carry

    lax.fori_loop(0, RPW // NBUF - 1, body, 0)
    for b in range(NBUF):
        drain_gather(b)
        scatter(b)

    @pl.when(w == NW - 1)
    def _():
        for k in range(XROWS):
            load_and_fire(0, NW * RPW + k)
            drain_gather(0)
            scatter(0)

    plsc.subcore_barrier()
    pltpu.sync_copy(acc.at[pl.ds(s * APT, APT)],
                    s_out.at[c, pl.ds(s * APT, APT)])


_BM = 1024
_GRID = NPAD // _BM
_SUB = _BM // D


def _dinv_rowscale(degp_ref):
    ones = jnp.ones((1, D), jnp.float32)
    parts = []
    for g in range(_SUB):
        deg = degp_ref[g, 0, :] + degp_ref[g, 1, :] + 1.0
        dinv = lax.rsqrt(jnp.maximum(deg, 1.0))
        parts.append(lax.dot_general(
            dinv.reshape(1, D), ones,
            (((0,), (0,)), ((), ())),
            preferred_element_type=jnp.float32,
            precision=lax.Precision.HIGHEST))
    return jnp.concatenate(parts, axis=0)


def _tc_first_body(x_ref, w_ref, degp_ref, g_ref):
    dmat = _dinv_rowscale(degp_ref)
    h = jnp.dot(x_ref[...], w_ref[...],
                preferred_element_type=jnp.float32,
                precision=lax.Precision.HIGHEST)
    g_ref[...] = h * dmat


def _tc_mid_body(sp_ref, g_ref, degp_ref, w_ref, b_ref, g2_ref):
    dmat = _dinv_rowscale(degp_ref)
    ssum = sp_ref[0] + sp_ref[1] + g_ref[...]
    h = jnp.maximum(ssum * dmat + b_ref[0, :], 0.0)
    g2_ref[...] = jnp.dot(h, w_ref[...],
                          preferred_element_type=jnp.float32,
                          precision=lax.Precision.HIGHEST) * dmat


def _tc_last_body(sp_ref, g_ref, degp_ref, b_ref, out_ref):
    dmat = _dinv_rowscale(degp_ref)
    ssum = sp_ref[0] + sp_ref[1] + g_ref[...]
    out_ref[...] = ssum * dmat + b_ref[0, :]


_BLK = pl.BlockSpec((_BM, D), lambda i: (i, 0))
_BLK_SP = pl.BlockSpec((NC, _BM, D), lambda i: (0, i, 0))
_BLK_DEG = pl.BlockSpec((_SUB, NC, D), lambda i: (i, 0, 0))
_BLK_W = pl.BlockSpec((D, D), lambda i: (0, 0))
_BLK_B = pl.BlockSpec((1, D), lambda i: (0, 0))
_OUT2D = jax.ShapeDtypeStruct((N, D), jnp.float32)

_tc_first = pl.pallas_call(
    _tc_first_body, grid=(_GRID,),
    in_specs=[_BLK, _BLK_W, _BLK_DEG],
    out_specs=_BLK, out_shape=_OUT2D)

_tc_mid = pl.pallas_call(
    _tc_mid_body, grid=(_GRID,),
    in_specs=[_BLK_SP, _BLK, _BLK_DEG, _BLK_W, _BLK_B],
    out_specs=_BLK, out_shape=_OUT2D)

_tc_last = pl.pallas_call(
    _tc_last_body, grid=(_GRID,),
    in_specs=[_BLK_SP, _BLK, _BLK_DEG, _BLK_B],
    out_specs=_BLK, out_shape=_OUT2D)


def kernel(x, edge_index, W1, b1, W2, b2):
    edge2d = edge_index.reshape(2, EROWS, CH)
    zeros1 = jnp.zeros((NPAD,), jnp.float32)
    zeros2 = jnp.zeros((NACC, D), jnp.float32)

    degp = _deg_kernel(edge2d, zeros1)
    degp3 = degp.reshape(NC, NPAD // D, D).transpose(1, 0, 2)

    g1 = _tc_first(x, W1, degp3)
    s1p = _agg_kernel(g1, edge2d, zeros2)
    g2 = _tc_mid(s1p, g1, degp3, W2, b1.reshape(1, D))
    s2p = _agg_kernel(g2, edge2d, zeros2)
    return _tc_last(s2p, g2, degp3, b2.reshape(1, D))


# speedup vs baseline: 33.3766x; 1.0651x over previous
import functools

import jax
import jax.numpy as jnp
from jax import lax
from jax.experimental import pallas as pl
from jax.experimental.pallas import tpu as pltpu
from jax.experimental.pallas import tpu_sc as plsc

N = 10000
D = 128
NPAD = 10240
E = 320000
NC, NS = 2, 16
NW = NC * NS
CH = 128
EROWS = E // CH
RPW = EROWS // NW
XROWS = EROWS - RPW * NW
RPT = NPAD // NS
NACC = 10112
APT = NACC // NS

_MESH = plsc.VectorSubcoreMesh(core_axis_name="c", subcore_axis_name="s")


KD = 6
DG = RPW // KD
assert DG * KD == RPW


@functools.partial(
    pl.kernel,
    out_type=jax.ShapeDtypeStruct((NPAD // D, NC, D), jnp.float32),
    mesh=_MESH,
    scratch_types=[
        pltpu.VMEM_SHARED((NPAD,), jnp.float32),
        pltpu.VMEM((2, KD, CH), jnp.int32),
        pltpu.VMEM((CH,), jnp.float32),
        pltpu.SemaphoreType.DMA,
        pltpu.SemaphoreType.DMA,
        pltpu.SemaphoreType.DMA,
        pltpu.SemaphoreType.DMA,
    ],
)
def _deg_kernel(edge_hbm, zeros1_hbm, deg_out, deg_acc, dst_v, ones_v,
                dsem0, dsem1, lsem0, lsem1):
    c = lax.axis_index("c")
    s = lax.axis_index("s")
    w = c * NS + s
    dsems = (dsem0, dsem1)
    lsems = (lsem0, lsem1)
    pltpu.sync_copy(zeros1_hbm.at[pl.ds(s * RPT, RPT)],
                    deg_acc.at[pl.ds(s * RPT, RPT)])
    for i in range(CH // 16):
        ones_v[pl.ds(i * 16, 16)] = jnp.ones((16,), jnp.float32)
    base = w * RPW

    def load_idx(b, grp):
        for j in range(KD):
            pltpu.async_copy(edge_hbm.at[1, base + grp * KD + j],
                             dst_v.at[b, j], lsems[b])

    def drain_idx(b, grp):
        for j in range(KD):
            pltpu.make_async_copy(edge_hbm.at[1, base + grp * KD + j],
                                  dst_v.at[b, j], lsems[b]).wait()

    def scatter_ones(b, grp):
        drain_idx(b, grp)
        descs = [pltpu.async_copy(ones_v, deg_acc.at[dst_v.at[b, j]],
                                  dsems[b], add=True)
                 for j in range(KD)]
        for d in descs:
            d.wait()

    plsc.subcore_barrier()
    for b in range(2):
        load_idx(b, b)

    def body(i, carry):
        for b in range(2):
            scatter_ones(b, i * 2 + b)
            load_idx(b, i * 2 + b + 2)
        return carry

    lax.fori_loop(0, DG // 2 - 1, body, 0)
    scatter_ones(0, DG - 3)
    load_idx(0, DG - 1)
    scatter_ones(1, DG - 2)
    scatter_ones(0, DG - 1)

    @pl.when(w == NW - 1)
    def _():
        for k in range(XROWS):
            pltpu.sync_copy(edge_hbm.at[1, NW * RPW + k], dst_v.at[0, 0])
            pltpu.sync_copy(ones_v, deg_acc.at[dst_v.at[0, 0]], add=True)

    plsc.subcore_barrier()
    for r in range(RPT // D):
        row = s * (RPT // D) + r
        pltpu.sync_copy(deg_acc.at[pl.ds(row * D, D)], deg_out.at[row, c])


NBUF = 3


@functools.partial(
    pl.kernel,
    out_type=jax.ShapeDtypeStruct((NC, NACC, D), jnp.float32),
    mesh=_MESH,
    scratch_types=[
        pltpu.VMEM_SHARED((NACC, D), jnp.float32),
        pltpu.VMEM((NBUF, 2, CH), jnp.int32),
        pltpu.VMEM((NBUF, CH, D), jnp.float32),
        pltpu.SemaphoreType.DMA,
        pltpu.SemaphoreType.DMA,
        pltpu.SemaphoreType.DMA,
        pltpu.SemaphoreType.DMA,
        pltpu.SemaphoreType.DMA,
        pltpu.SemaphoreType.DMA,
    ],
)
def _agg_kernel(g_hbm, edge_hbm, zeros2_hbm, s_out,
                acc, idx_v, rows_v, gsem0, gsem1, gsem2, ssem0, ssem1, ssem2):
    c = lax.axis_index("c")
    s = lax.axis_index("s")
    w = c * NS + s
    gsems = (gsem0, gsem1, gsem2)
    ssems = (ssem0, ssem1, ssem2)
    pltpu.sync_copy(zeros2_hbm.at[pl.ds(s * APT, APT)],
                    acc.at[pl.ds(s * APT, APT)])
    base = w * RPW

    def load_and_fire(b, chunk):
        pltpu.sync_copy(edge_hbm.at[:, chunk], idx_v.at[b])
        pltpu.async_copy(g_hbm.at[idx_v.at[b, 0]], rows_v.at[b], gsems[b])

    def drain_gather(b):
        pltpu.make_async_copy(g_hbm.at[idx_v.at[b, 0]],
                              rows_v.at[b], gsems[b]).wait()

    def fire_scatter(b):
        pltpu.async_copy(rows_v.at[b], acc.at[idx_v.at[b, 1]], ssems[b],
                         add=True)

    def drain_scatter(b):
        pltpu.make_async_copy(rows_v.at[b], acc.at[idx_v.at[b, 1]],
                              ssems[b]).wait()

    plsc.subcore_barrier()

    for b in range(2):
        load_and_fire(b, base + b)

    def step(r, chunk, first, fire_ahead):
        drain_gather(r)
        fire_scatter(r)
        if not first:
            drain_scatter((r - 1) % NBUF)
        if fire_ahead:
            load_and_fire((r + 2) % NBUF, chunk + 2)

    step(0, base, True, True)

    def body(i, carry):
        c0 = 1 + i * NBUF
        for k in range(NBUF):
            step((1 + k) % NBUF, base + c0 + k, False, True)
        return carry

    lax.fori_loop(0, (RPW - 3) // NBUF, body, 0)
    step((RPW - 2) % NBUF, base + RPW - 2, False, False)
    step((RPW - 1) % NBUF, base + RPW - 1, False, False)
    drain_scatter((RPW - 1) % NBUF)

    @pl.when(w == NW - 1)
    def _():
        for k in range(XROWS):
            load_and_fire(0, NW * RPW + k)
            drain_gather(0)
            fire_scatter(0)
            drain_scatter(0)

    plsc.subcore_barrier()
    pltpu.sync_copy(acc.at[pl.ds(s * APT, APT)],
                    s_out.at[c, pl.ds(s * APT, APT)])


_BM = 1024
_GRID = NPAD // _BM
_SUB = _BM // D


def _dinv_rowscale(degp_ref):
    ones = jnp.ones((1, D), jnp.float32)
    parts = []
    for g in range(_SUB):
        deg = degp_ref[g, 0, :] + degp_ref[g, 1, :] + 1.0
        dinv = lax.rsqrt(jnp.maximum(deg, 1.0))
        parts.append(lax.dot_general(
            dinv.reshape(1, D), ones,
            (((0,), (0,)), ((), ())),
            preferred_element_type=jnp.float32,
            precision=lax.Precision.HIGHEST))
    return jnp.concatenate(parts, axis=0)


def _tc_first_body(x_ref, w_ref, degp_ref, g_ref):
    dmat = _dinv_rowscale(degp_ref)
    h = jnp.dot(x_ref[...], w_ref[...],
                preferred_element_type=jnp.float32,
                precision=lax.Precision.HIGHEST)
    g_ref[...] = h * dmat


def _tc_mid_body(sp_ref, g_ref, degp_ref, w_ref, b_ref, g2_ref):
    dmat = _dinv_rowscale(degp_ref)
    ssum = sp_ref[0] + sp_ref[1] + g_ref[...]
    h = jnp.maximum(ssum * dmat + b_ref[0, :], 0.0)
    g2_ref[...] = jnp.dot(h, w_ref[...],
                          preferred_element_type=jnp.float32,
                          precision=lax.Precision.HIGHEST) * dmat


def _tc_last_body(sp_ref, g_ref, degp_ref, b_ref, out_ref):
    dmat = _dinv_rowscale(degp_ref)
    ssum = sp_ref[0] + sp_ref[1] + g_ref[...]
    out_ref[...] = ssum * dmat + b_ref[0, :]


_BLK = pl.BlockSpec((_BM, D), lambda i: (i, 0))
_BLK_SP = pl.BlockSpec((NC, _BM, D), lambda i: (0, i, 0))
_BLK_DEG = pl.BlockSpec((_SUB, NC, D), lambda i: (i, 0, 0))
_BLK_W = pl.BlockSpec((D, D), lambda i: (0, 0))
_BLK_B = pl.BlockSpec((1, D), lambda i: (0, 0))
_OUT2D = jax.ShapeDtypeStruct((N, D), jnp.float32)

_tc_first = pl.pallas_call(
    _tc_first_body, grid=(_GRID,),
    in_specs=[_BLK, _BLK_W, _BLK_DEG],
    out_specs=_BLK, out_shape=_OUT2D)

_tc_mid = pl.pallas_call(
    _tc_mid_body, grid=(_GRID,),
    in_specs=[_BLK_SP, _BLK, _BLK_DEG, _BLK_W, _BLK_B],
    out_specs=_BLK, out_shape=_OUT2D)

_tc_last = pl.pallas_call(
    _tc_last_body, grid=(_GRID,),
    in_specs=[_BLK_SP, _BLK, _BLK_DEG, _BLK_B],
    out_specs=_BLK, out_shape=_OUT2D)


def kernel(x, edge_index, W1, b1, W2, b2):
    edge2d = edge_index.reshape(2, EROWS, CH)
    zeros1 = jnp.zeros((NPAD,), jnp.float32)
    zeros2 = jnp.zeros((NACC, D), jnp.float32)

    degp3 = _deg_kernel(edge2d, zeros1)

    g1 = _tc_first(x, W1, degp3)
    s1p = _agg_kernel(g1, edge2d, zeros2)
    g2 = _tc_mid(s1p, g1, degp3, W2, b1.reshape(1, D))
    s2p = _agg_kernel(g2, edge2d, zeros2)
    return _tc_last(s2p, g2, degp3, b2.reshape(1, D))


# speedup vs baseline: 33.6680x; 1.0087x over previous
import functools

import jax
import jax.numpy as jnp
from jax import lax
from jax.experimental import pallas as pl
from jax.experimental.pallas import tpu as pltpu
from jax.experimental.pallas import tpu_sc as plsc

N = 10000
D = 128
NPAD = 10240
E = 320000
NC, NS = 2, 16
NW = NC * NS
CH = 128
EROWS = E // CH
RPW = EROWS // NW
XROWS = EROWS - RPW * NW
RPT = NPAD // NS
NACC = 10112
APT = NACC // NS

_MESH = plsc.VectorSubcoreMesh(core_axis_name="c", subcore_axis_name="s")


KD = 6
DG = RPW // KD
assert DG * KD == RPW


@functools.partial(
    pl.kernel,
    out_type=jax.ShapeDtypeStruct((NPAD // D, NC, D), jnp.float32),
    mesh=_MESH,
    scratch_types=[
        pltpu.VMEM_SHARED((NPAD,), jnp.float32),
        pltpu.VMEM((2, KD, CH), jnp.int32),
        pltpu.VMEM((CH,), jnp.float32),
        pltpu.SemaphoreType.DMA,
        pltpu.SemaphoreType.DMA,
        pltpu.SemaphoreType.DMA,
        pltpu.SemaphoreType.DMA,
    ],
)
def _deg_kernel(edge_hbm, zeros1_hbm, deg_out, deg_acc, dst_v, ones_v,
                dsem0, dsem1, lsem0, lsem1):
    c = lax.axis_index("c")
    s = lax.axis_index("s")
    w = c * NS + s
    dsems = (dsem0, dsem1)
    lsems = (lsem0, lsem1)
    pltpu.sync_copy(zeros1_hbm.at[pl.ds(s * RPT, RPT)],
                    deg_acc.at[pl.ds(s * RPT, RPT)])
    for i in range(CH // 16):
        ones_v[pl.ds(i * 16, 16)] = jnp.ones((16,), jnp.float32)
    base = w * RPW

    def load_idx(b, grp):
        for j in range(KD):
            pltpu.async_copy(edge_hbm.at[1, base + grp * KD + j],
                             dst_v.at[b, j], lsems[b])

    def drain_idx(b, grp):
        for j in range(KD):
            pltpu.make_async_copy(edge_hbm.at[1, base + grp * KD + j],
                                  dst_v.at[b, j], lsems[b]).wait()

    def scatter_ones(b, grp):
        drain_idx(b, grp)
        descs = [pltpu.async_copy(ones_v, deg_acc.at[dst_v.at[b, j]],
                                  dsems[b], add=True)
                 for j in range(KD)]
        for d in descs:
            d.wait()

    plsc.subcore_barrier()
    for b in range(2):
        load_idx(b, b)

    def body(i, carry):
        for b in range(2):
            scatter_ones(b, i * 2 + b)
            load_idx(b, i * 2 + b + 2)
        return carry

    lax.fori_loop(0, DG // 2 - 1, body, 0)
    scatter_ones(0, DG - 3)
    load_idx(0, DG - 1)
    scatter_ones(1, DG - 2)
    scatter_ones(0, DG - 1)

    @pl.when(w == NW - 1)
    def _():
        for k in range(XROWS):
            pltpu.sync_copy(edge_hbm.at[1, NW * RPW + k], dst_v.at[0, 0])
            pltpu.sync_copy(ones_v, deg_acc.at[dst_v.at[0, 0]], add=True)

    plsc.subcore_barrier()
    for r in range(RPT // D):
        row = s * (RPT // D) + r
        pltpu.sync_copy(deg_acc.at[pl.ds(row * D, D)], deg_out.at[row, c])


NBUF = 3


@functools.partial(
    pl.kernel,
    out_type=jax.ShapeDtypeStruct((NC, NACC, D), jnp.float32),
    mesh=_MESH,
    scratch_types=[
        pltpu.VMEM_SHARED((NACC, D), jnp.float32),
        pltpu.VMEM((NBUF, 2, CH), jnp.int32),
        pltpu.VMEM((NBUF, CH, D), jnp.float32),
        pltpu.SemaphoreType.DMA,
        pltpu.SemaphoreType.DMA,
        pltpu.SemaphoreType.DMA,
        pltpu.SemaphoreType.DMA,
        pltpu.SemaphoreType.DMA,
        pltpu.SemaphoreType.DMA,
    ],
)
def _agg_kernel(g_hbm, edge_hbm, zeros2_hbm, s_out,
                acc, idx_v, rows_v, gsem0, gsem1, gsem2, ssem0, ssem1, ssem2):
    c = lax.axis_index("c")
    s = lax.axis_index("s")
    w = c * NS + s
    gsems = (gsem0, gsem1, gsem2)
    ssems = (ssem0, ssem1, ssem2)
    pltpu.sync_copy(zeros2_hbm.at[pl.ds(s * APT, APT)],
                    acc.at[pl.ds(s * APT, APT)])
    base = w * RPW

    def load_and_fire(b, chunk):
        pltpu.sync_copy(edge_hbm.at[:, chunk], idx_v.at[b])
        pltpu.async_copy(g_hbm.at[idx_v.at[b, 0]], rows_v.at[b], gsems[b])

    def drain_gather(b):
        pltpu.make_async_copy(g_hbm.at[idx_v.at[b, 0]],
                              rows_v.at[b], gsems[b]).wait()

    def fire_scatter(b):
        pltpu.async_copy(rows_v.at[b], acc.at[idx_v.at[b, 1]], ssems[b],
                         add=True)

    def drain_scatter(b):
        pltpu.make_async_copy(rows_v.at[b], acc.at[idx_v.at[b, 1]],
                              ssems[b]).wait()

    plsc.subcore_barrier()

    for b in range(2):
        load_and_fire(b, base + b)

    def step(r, chunk, first, fire_ahead):
        drain_gather(r)
        fire_scatter(r)
        if not first:
            drain_scatter((r - 1) % NBUF)
        if fire_ahead:
            load_and_fire((r + 2) % NBUF, chunk + 2)

    step(0, base, True, True)

    def body(i, carry):
        c0 = 1 + i * NBUF
        for k in range(NBUF):
            step((1 + k) % NBUF, base + c0 + k, False, True)
        return carry

    lax.fori_loop(0, (RPW - 3) // NBUF, body, 0)
    step((RPW - 2) % NBUF, base + RPW - 2, False, False)
    step((RPW - 1) % NBUF, base + RPW - 1, False, False)
    drain_scatter((RPW - 1) % NBUF)

    @pl.when(w == NW - 1)
    def _():
        for k in range(XROWS):
            load_and_fire(0, NW * RPW + k)
            drain_gather(0)
            fire_scatter(0)
            drain_scatter(0)

    plsc.subcore_barrier()
    pltpu.sync_copy(acc.at[pl.ds(s * APT, APT)],
                    s_out.at[c, pl.ds(s * APT, APT)])


_BM = 1024
_GRID = NPAD // _BM
_SUB = _BM // D


def _dinv_rowscale(degp_ref):
    ones = jnp.ones((1, D), jnp.float32)
    parts = []
    for g in range(_SUB):
        deg = degp_ref[g, 0, :] + degp_ref[g, 1, :] + 1.0
        dinv = lax.rsqrt(jnp.maximum(deg, 1.0))
        parts.append(lax.dot_general(
            dinv.reshape(1, D), ones,
            (((0,), (0,)), ((), ())),
            preferred_element_type=jnp.float32,
            precision=lax.Precision.HIGHEST))
    return jnp.concatenate(parts, axis=0)


def _tc_mm_body(x_ref, w_ref, h_ref):
    h_ref[...] = jnp.dot(x_ref[...], w_ref[...],
                         preferred_element_type=jnp.float32,
                         precision=lax.Precision.HIGHEST)


def _tc_scale_body(h_ref, degp_ref, g_ref):
    g_ref[...] = h_ref[...] * _dinv_rowscale(degp_ref)


def _tc_mid_body(sp_ref, g_ref, degp_ref, w_ref, b_ref, g2_ref):
    dmat = _dinv_rowscale(degp_ref)
    ssum = sp_ref[0] + sp_ref[1] + g_ref[...]
    h = jnp.maximum(ssum * dmat + b_ref[0, :], 0.0)
    g2_ref[...] = jnp.dot(h, w_ref[...],
                          preferred_element_type=jnp.float32,
                          precision=lax.Precision.HIGHEST) * dmat


def _tc_last_body(sp_ref, g_ref, degp_ref, b_ref, out_ref):
    dmat = _dinv_rowscale(degp_ref)
    ssum = sp_ref[0] + sp_ref[1] + g_ref[...]
    out_ref[...] = ssum * dmat + b_ref[0, :]


_BLK = pl.BlockSpec((_BM, D), lambda i: (i, 0))
_BLK_SP = pl.BlockSpec((NC, _BM, D), lambda i: (0, i, 0))
_BLK_DEG = pl.BlockSpec((_SUB, NC, D), lambda i: (i, 0, 0))
_BLK_W = pl.BlockSpec((D, D), lambda i: (0, 0))
_BLK_B = pl.BlockSpec((1, D), lambda i: (0, 0))
_OUT2D = jax.ShapeDtypeStruct((N, D), jnp.float32)

_tc_mm = pl.pallas_call(
    _tc_mm_body, grid=(_GRID,),
    in_specs=[_BLK, _BLK_W],
    out_specs=_BLK, out_shape=_OUT2D)

_tc_scale = pl.pallas_call(
    _tc_scale_body, grid=(_GRID,),
    in_specs=[_BLK, _BLK_DEG],
    out_specs=_BLK, out_shape=_OUT2D)

_tc_mid = pl.pallas_call(
    _tc_mid_body, grid=(_GRID,),
    in_specs=[_BLK_SP, _BLK, _BLK_DEG, _BLK_W, _BLK_B],
    out_specs=_BLK, out_shape=_OUT2D)

_tc_last = pl.pallas_call(
    _tc_last_body, grid=(_GRID,),
    in_specs=[_BLK_SP, _BLK, _BLK_DEG, _BLK_B],
    out_specs=_BLK, out_shape=_OUT2D)


def kernel(x, edge_index, W1, b1, W2, b2):
    edge2d = edge_index.reshape(2, EROWS, CH)
    zeros1 = jnp.zeros((NPAD,), jnp.float32)
    zeros2 = jnp.zeros((NACC, D), jnp.float32)

    degp3 = _deg_kernel(edge2d, zeros1)

    h1 = _tc_mm(x, W1)
    g1 = _tc_scale(h1, degp3)
    s1p = _agg_kernel(g1, edge2d, zeros2)
    g2 = _tc_mid(s1p, g1, degp3, W2, b1.reshape(1, D))
    s2p = _agg_kernel(g2, edge2d, zeros2)
    return _tc_last(s2p, g2, degp3, b2.reshape(1, D))
